# baseline (speedup 1.0000x reference)
import jax
import jax.numpy as jnp
from jax.experimental import pallas as pl
from jax.experimental.pallas import tpu as pltpu

_BM = 200


def _gcn2_block(scal_ref, adj_a_ref, adj_b_ref, x_ref, h0_ref, w_ref, out_ref):
    alpha = scal_ref[0]
    theta = scal_ref[1]
    w = w_ref[...]
    x = x_ref[...]
    for h, adj_ref in enumerate((adj_a_ref, adj_b_ref)):
        hi = jnp.dot(adj_ref[0], x, preferred_element_type=jnp.float32)
        support = (1.0 - alpha) * hi + alpha * h0_ref[h]
        out_ref[h] = (
            theta * jnp.dot(support, w, preferred_element_type=jnp.float32)
            + (1.0 - theta) * support
        )


def kernel(input, adj, adj_high, h0, lamda, alpha, l, weight_low):
    n, d = input.shape
    half = n // 2
    theta = jnp.log(lamda / l + 1.0)
    scal = jnp.stack([alpha.astype(jnp.float32), theta.astype(jnp.float32)])
    adj3 = adj.reshape(2, half, n)
    h03 = h0.reshape(2, half, d)
    out = pl.pallas_call(
        _gcn2_block,
        grid=(half // _BM,),
        in_specs=[
            pl.BlockSpec(memory_space=pltpu.SMEM),
            pl.BlockSpec((1, _BM, n), lambda i: (0, i, 0)),
            pl.BlockSpec((1, _BM, n), lambda i: (1, i, 0)),
            pl.BlockSpec((n, d), lambda i: (0, 0)),
            pl.BlockSpec((2, _BM, d), lambda i: (0, i, 0)),
            pl.BlockSpec((d, d), lambda i: (0, 0)),
        ],
        out_specs=pl.BlockSpec((2, _BM, d), lambda i: (0, i, 0)),
        out_shape=jax.ShapeDtypeStruct((2, half, d), jnp.float32),
        compiler_params=pltpu.CompilerParams(
            dimension_semantics=("arbitrary",),
        ),
    )(scal, adj3, adj3, input, h03, weight_low)
    return out.reshape(n, d)


# speedup vs baseline: 1.0513x; 1.0513x over previous
import jax
import jax.numpy as jnp
from jax.experimental import pallas as pl
from jax.experimental.pallas import tpu as pltpu

_BM = 512


def _gcn2_block(scal_ref, adj_ref, x_ref, h0_ref, w_ref, out_ref):
    alpha = scal_ref[0]
    theta = scal_ref[1]
    hi = jnp.dot(adj_ref[...], x_ref[...], preferred_element_type=jnp.float32)
    support = (1.0 - alpha) * hi + alpha * h0_ref[...]
    out_ref[...] = (
        theta * jnp.dot(support, w_ref[...], preferred_element_type=jnp.float32)
        + (1.0 - theta) * support
    )


def kernel(input, adj, adj_high, h0, lamda, alpha, l, weight_low):
    n, d = input.shape
    theta = jnp.log(lamda / l + 1.0)
    scal = jnp.stack([alpha.astype(jnp.float32), theta.astype(jnp.float32)])
    return pl.pallas_call(
        _gcn2_block,
        grid=(pl.cdiv(n, _BM),),
        in_specs=[
            pl.BlockSpec(memory_space=pltpu.SMEM),
            pl.BlockSpec((_BM, n), lambda i: (i, 0)),
            pl.BlockSpec((n, d), lambda i: (0, 0)),
            pl.BlockSpec((_BM, d), lambda i: (i, 0)),
            pl.BlockSpec((d, d), lambda i: (0, 0)),
        ],
        out_specs=pl.BlockSpec((_BM, d), lambda i: (i, 0)),
        out_shape=jax.ShapeDtypeStruct((n, d), jnp.float32),
        compiler_params=pltpu.CompilerParams(
            dimension_semantics=("arbitrary",),
        ),
    )(scal, adj, input, h0, weight_low)
